# Initial kernel scaffold; baseline (speedup 1.0000x reference)
import jax
import jax.numpy as jnp
from jax.experimental import pallas as pl


def kernel(x, y):
    raise NotImplementedError("write your pallas kernel here")


# speedup vs baseline: 95.0074x; 95.0074x over previous
import functools

import jax
import jax.numpy as jnp
import numpy as np
from jax import lax
from jax.experimental import pallas as pl
from jax.experimental.pallas import tpu as pltpu
from jax.experimental.pallas import tpu_sc as plsc

B = 32
N = 2048
D = 3
L = 16
NV = N // L

THETA = float(np.float32(-1.0 / np.sqrt(3.0)))


def _dir_sort(block, asc):
    key = jnp.where(asc, block, -block)
    skey, _ = plsc.sort_key_val(key, key)
    return jnp.where(asc, skey, -skey)


def _sort_seq(buf):
    def init_body(i, c):
        asc = (i & 1) == 0
        buf[pl.ds(i * L, L)] = _dir_sort(buf[pl.ds(i * L, L)], asc)
        return c

    lax.fori_loop(0, NV, init_body, 0)

    m = 32
    while m <= N:
        mv = m // L
        d = m // 2
        while d >= L:
            dv = d // L
            sh = dv.bit_length() - 1

            def ce_body(j, c, dv=dv, sh=sh, mv=mv):
                v = ((j >> sh) << (sh + 1)) | (j & (dv - 1))
                asc = (v & mv) == 0
                a = buf[pl.ds(v * L, L)]
                b = buf[pl.ds((v + dv) * L, L)]
                mn = jnp.minimum(a, b)
                mx = jnp.maximum(a, b)
                buf[pl.ds(v * L, L)] = jnp.where(asc, mn, mx)
                buf[pl.ds((v + dv) * L, L)] = jnp.where(asc, mx, mn)
                return c

            lax.fori_loop(0, NV // 2, ce_body, 0)
            d //= 2

        def fin_body(i, c, mv=mv):
            asc = (i & mv) == 0
            buf[pl.ds(i * L, L)] = _dir_sort(buf[pl.ds(i * L, L)], asc)
            return c

        lax.fori_loop(0, NV, fin_body, 0)
        m *= 2


def _project(raw, out):
    def body(i, c):
        p0 = raw[pl.ds(i * L, L)]
        p1 = raw[pl.ds(N + i * L, L)]
        p2 = raw[pl.ds(2 * N + i * L, L)]
        out[pl.ds(i * L, L)] = (p0 + p1 + p2) * THETA
        return c

    lax.fori_loop(0, NV, body, 0)


_SC_MESH = plsc.VectorSubcoreMesh(core_axis_name="c", subcore_axis_name="s")


@functools.partial(
    pl.kernel,
    out_type=jax.ShapeDtypeStruct((B, L), jnp.float32),
    mesh=_SC_MESH,
    compiler_params=pltpu.CompilerParams(needs_layout_passes=False),
    scratch_types=[
        pltpu.VMEM((N * D,), jnp.float32),
        pltpu.VMEM((N * D,), jnp.float32),
        pltpu.VMEM((N,), jnp.float32),
        pltpu.VMEM((N,), jnp.float32),
        pltpu.VMEM((L,), jnp.float32),
    ],
)
def _swd_sc(x_hbm, y_hbm, out_hbm, xraw, yraw, u, v, ovec):
    b = lax.axis_index("s") * 2 + lax.axis_index("c")

    pltpu.sync_copy(x_hbm.at[b], xraw)
    pltpu.sync_copy(y_hbm.at[b], yraw)

    _project(xraw, u)
    _project(yraw, v)
    _sort_seq(u)
    _sort_seq(v)

    def diff_body(i, acc):
        du = u[pl.ds(i * L, L)] - v[pl.ds(i * L, L)]
        return acc + du * du

    acc = lax.fori_loop(0, NV, diff_body, jnp.zeros((L,), jnp.float32))
    s = jnp.sum(acc)
    ovec[...] = jnp.broadcast_to(s, (L,))
    pltpu.sync_copy(ovec, out_hbm.at[b])


def _finish_tc(s_ref, o_ref):
    o_ref[0, 0] = jnp.sum(jnp.sqrt(s_ref[...])) * jnp.float32(1.0 / (B * L))


_finish = pl.pallas_call(
    _finish_tc,
    out_shape=jax.ShapeDtypeStruct((1, 1), jnp.float32),
    out_specs=pl.BlockSpec(memory_space=pltpu.SMEM),
)


def kernel(x, y):
    xf = x.transpose(0, 2, 1).reshape(B, D * N)
    yf = y.transpose(0, 2, 1).reshape(B, D * N)
    s = _swd_sc(xf, yf)
    return _finish(s)[0, 0]


# speedup vs baseline: 203.8953x; 2.1461x over previous
import functools

import jax
import jax.numpy as jnp
import numpy as np
from jax import lax
from jax.experimental import pallas as pl
from jax.experimental.pallas import tpu as pltpu
from jax.experimental.pallas import tpu_sc as plsc

B = 32
N = 2048
D = 3
L = 16
NV = N // L

THETA = float(np.float32(-1.0 / np.sqrt(3.0)))


def _vsort(a):
    sk, _ = plsc.sort_key_val(a, a)
    return sk


def _sort_two(u, v):
    m = 16
    while m < N:
        mv2 = (2 * m) // L
        hv = m // L
        sh = hv.bit_length() - 1

        def rev_body(i, c, sh=sh, hv=hv, mv2=mv2):
            for k in range(2):
                j = i * 2 + k
                r = j >> sh
                jj = j & (hv - 1)
                vi = r * mv2 + jj
                wi = r * mv2 + (mv2 - 1) - jj
                for buf in (u, v):
                    a = buf[pl.ds(vi * L, L)]
                    b = lax.rev(buf[pl.ds(wi * L, L)], (0,))
                    buf[pl.ds(vi * L, L)] = jnp.minimum(a, b)
                    buf[pl.ds(wi * L, L)] = lax.rev(jnp.maximum(a, b), (0,))
            return c

        lax.fori_loop(0, NV // 4, rev_body, 0)

        d = m // 2
        while d >= L:
            dv = d // L
            sd = dv.bit_length() - 1

            def ce_body(i, c, dv=dv, sd=sd):
                for k in range(4):
                    j = i * 4 + k
                    vi = ((j >> sd) << (sd + 1)) | (j & (dv - 1))
                    for buf in (u, v):
                        a = buf[pl.ds(vi * L, L)]
                        b = buf[pl.ds((vi + dv) * L, L)]
                        buf[pl.ds(vi * L, L)] = jnp.minimum(a, b)
                        buf[pl.ds((vi + dv) * L, L)] = jnp.maximum(a, b)
                return c

            lax.fori_loop(0, NV // 8, ce_body, 0)
            d //= 2

        def fin_body(i, c):
            for k in range(4):
                j = i * 4 + k
                for buf in (u, v):
                    buf[pl.ds(j * L, L)] = _vsort(buf[pl.ds(j * L, L)])
            return c

        lax.fori_loop(0, NV // 4, fin_body, 0)
        m *= 2


_SC_MESH = plsc.VectorSubcoreMesh(core_axis_name="c", subcore_axis_name="s")


@functools.partial(
    pl.kernel,
    out_type=jax.ShapeDtypeStruct((B, L), jnp.float32),
    mesh=_SC_MESH,
    compiler_params=pltpu.CompilerParams(needs_layout_passes=False),
    scratch_types=[
        pltpu.VMEM((N * D,), jnp.float32),
        pltpu.VMEM((N * D,), jnp.float32),
        pltpu.VMEM((N,), jnp.float32),
        pltpu.VMEM((N,), jnp.float32),
        pltpu.VMEM((L,), jnp.float32),
    ],
)
def _swd_sc(x_hbm, y_hbm, out_hbm, xraw, yraw, u, v, ovec):
    b = lax.axis_index("s") * 2 + lax.axis_index("c")

    pltpu.sync_copy(x_hbm.at[b], xraw)
    pltpu.sync_copy(y_hbm.at[b], yraw)

    def proj_body(i, c):
        for k in range(2):
            j = i * 2 + k
            for raw, out in ((xraw, u), (yraw, v)):
                p0 = raw[pl.ds(j * L, L)]
                p1 = raw[pl.ds(N + j * L, L)]
                p2 = raw[pl.ds(2 * N + j * L, L)]
                out[pl.ds(j * L, L)] = _vsort((p0 + p1 + p2) * THETA)
        return c

    lax.fori_loop(0, NV // 2, proj_body, 0)

    _sort_two(u, v)

    def diff_body(i, acc):
        a0, a1 = acc
        d0 = u[pl.ds((2 * i) * L, L)] - v[pl.ds((2 * i) * L, L)]
        d1 = u[pl.ds((2 * i + 1) * L, L)] - v[pl.ds((2 * i + 1) * L, L)]
        return (a0 + d0 * d0, a1 + d1 * d1)

    zero = jnp.zeros((L,), jnp.float32)
    acc0, acc1 = lax.fori_loop(0, NV // 2, diff_body, (zero, zero))
    s = jnp.sum(acc0 + acc1)
    ovec[...] = jnp.broadcast_to(s, (L,))
    pltpu.sync_copy(ovec, out_hbm.at[b])


def _finish_tc(s_ref, o_ref):
    o_ref[0, 0] = jnp.sum(jnp.sqrt(s_ref[...])) * jnp.float32(1.0 / (B * L))


_finish = pl.pallas_call(
    _finish_tc,
    out_shape=jax.ShapeDtypeStruct((1, 1), jnp.float32),
    out_specs=pl.BlockSpec(memory_space=pltpu.SMEM),
)


def kernel(x, y):
    xf = x.transpose(0, 2, 1).reshape(B, D * N)
    yf = y.transpose(0, 2, 1).reshape(B, D * N)
    s = _swd_sc(xf, yf)
    return _finish(s)[0, 0]


# speedup vs baseline: 213.1812x; 1.0455x over previous
import functools

import jax
import jax.numpy as jnp
import numpy as np
from jax import lax
from jax.experimental import pallas as pl
from jax.experimental.pallas import tpu as pltpu
from jax.experimental.pallas import tpu_sc as plsc

B = 32
N = 2048
D = 3
L = 16
NV = N // L

THETA = float(np.float32(-1.0 / np.sqrt(3.0)))


def _vsort(a):
    sk, _ = plsc.sort_key_val(a, a)
    return sk


def _sort_two(u, v):
    m = 16
    while m < N:
        mv2 = (2 * m) // L
        hv = m // L
        sh = hv.bit_length() - 1

        @plsc.parallel_loop(0, NV // 2, unroll=4)
        def rev_body(j, sh=sh, hv=hv, mv2=mv2):
            r = j >> sh
            jj = j & (hv - 1)
            vi = r * mv2 + jj
            wi = r * mv2 + (mv2 - 1) - jj
            for buf in (u, v):
                a = buf[pl.ds(vi * L, L)]
                b = lax.rev(buf[pl.ds(wi * L, L)], (0,))
                buf[pl.ds(vi * L, L)] = jnp.minimum(a, b)
                buf[pl.ds(wi * L, L)] = lax.rev(jnp.maximum(a, b), (0,))

        d = m // 2
        while d >= L:
            dv = d // L
            sd = dv.bit_length() - 1

            @plsc.parallel_loop(0, NV // 2, unroll=4)
            def ce_body(j, dv=dv, sd=sd):
                vi = ((j >> sd) << (sd + 1)) | (j & (dv - 1))
                for buf in (u, v):
                    a = buf[pl.ds(vi * L, L)]
                    b = buf[pl.ds((vi + dv) * L, L)]
                    buf[pl.ds(vi * L, L)] = jnp.minimum(a, b)
                    buf[pl.ds((vi + dv) * L, L)] = jnp.maximum(a, b)

            d //= 2

        @plsc.parallel_loop(0, NV, unroll=4)
        def fin_body(j):
            for buf in (u, v):
                buf[pl.ds(j * L, L)] = _vsort(buf[pl.ds(j * L, L)])

        m *= 2


_SC_MESH = plsc.VectorSubcoreMesh(core_axis_name="c", subcore_axis_name="s")


@functools.partial(
    pl.kernel,
    out_type=jax.ShapeDtypeStruct((B, L), jnp.float32),
    mesh=_SC_MESH,
    compiler_params=pltpu.CompilerParams(needs_layout_passes=False),
    scratch_types=[
        pltpu.VMEM((N * D,), jnp.float32),
        pltpu.VMEM((N * D,), jnp.float32),
        pltpu.VMEM((N,), jnp.float32),
        pltpu.VMEM((N,), jnp.float32),
        pltpu.VMEM((L,), jnp.float32),
    ],
)
def _swd_sc(x_hbm, y_hbm, out_hbm, xraw, yraw, u, v, ovec):
    b = lax.axis_index("s") * 2 + lax.axis_index("c")

    pltpu.sync_copy(x_hbm.at[b], xraw)
    pltpu.sync_copy(y_hbm.at[b], yraw)

    @plsc.parallel_loop(0, NV, unroll=4)
    def proj_body(j):
        for raw, out in ((xraw, u), (yraw, v)):
            p0 = raw[pl.ds(j * L, L)]
            p1 = raw[pl.ds(N + j * L, L)]
            p2 = raw[pl.ds(2 * N + j * L, L)]
            out[pl.ds(j * L, L)] = _vsort((p0 + p1 + p2) * THETA)

    _sort_two(u, v)

    zero = jnp.zeros((L,), jnp.float32)

    @plsc.parallel_loop(0, NV // 2, unroll=2, carry=(zero, zero))
    def diff_acc(i, acc):
        a0, a1 = acc
        d0 = u[pl.ds((2 * i) * L, L)] - v[pl.ds((2 * i) * L, L)]
        d1 = u[pl.ds((2 * i + 1) * L, L)] - v[pl.ds((2 * i + 1) * L, L)]
        return (a0 + d0 * d0, a1 + d1 * d1)

    acc0, acc1 = diff_acc
    s = jnp.sum(acc0 + acc1)
    ovec[...] = jnp.broadcast_to(s, (L,))
    pltpu.sync_copy(ovec, out_hbm.at[b])


def _finish_tc(s_ref, o_ref):
    o_ref[0, 0] = jnp.sum(jnp.sqrt(s_ref[...])) * jnp.float32(1.0 / (B * L))


_finish = pl.pallas_call(
    _finish_tc,
    out_shape=jax.ShapeDtypeStruct((1, 1), jnp.float32),
    out_specs=pl.BlockSpec(memory_space=pltpu.SMEM),
)


def kernel(x, y):
    xf = x.transpose(0, 2, 1).reshape(B, D * N)
    yf = y.transpose(0, 2, 1).reshape(B, D * N)
    s = _swd_sc(xf, yf)
    return _finish(s)[0, 0]


# speedup vs baseline: 253.2239x; 1.1878x over previous
import functools

import jax
import jax.numpy as jnp
import numpy as np
from jax import lax
from jax.experimental import pallas as pl
from jax.experimental.pallas import tpu as pltpu
from jax.experimental.pallas import tpu_sc as plsc

B = 32
N = 2048
D = 3
L = 16
NV = N // L

THETA = float(np.float32(-1.0 / np.sqrt(3.0)))


def _vsort(a):
    sk, _ = plsc.sort_key_val(a, a)
    return sk


def _apply_group(regs, stages):
    G = len(regs)
    for st in stages:
        if st == "R":
            for i in range(G // 2):
                a = regs[i]
                b = lax.rev(regs[G - 1 - i], (0,))
                regs[i] = jnp.minimum(a, b)
                regs[G - 1 - i] = lax.rev(jnp.maximum(a, b), (0,))
        elif st == "s":
            for i in range(G):
                regs[i] = _vsort(regs[i])
        else:
            dv = st[1]
            for i in range(G):
                if (i & dv) == 0:
                    a, b = regs[i], regs[i + dv]
                    regs[i] = jnp.minimum(a, b)
                    regs[i + dv] = jnp.maximum(a, b)
    return regs


def _fused_pass(bufs, G, stages, unroll=1):
    @plsc.parallel_loop(0, NV // G, unroll=unroll)
    def body(g):
        g0 = g * G
        for buf in bufs:
            regs = [buf[pl.ds((g0 + i) * L, L)] for i in range(G)]
            regs = _apply_group(regs, stages)
            for i in range(G):
                buf[pl.ds((g0 + i) * L, L)] = regs[i]


def _rev_pass(bufs, mv2):
    hv = mv2 // 2
    sh = hv.bit_length() - 1

    @plsc.parallel_loop(0, NV // 2, unroll=4)
    def body(j):
        r = j >> sh
        jj = j & (hv - 1)
        vi = r * mv2 + jj
        wi = r * mv2 + (mv2 - 1) - jj
        for buf in bufs:
            a = buf[pl.ds(vi * L, L)]
            b = lax.rev(buf[pl.ds(wi * L, L)], (0,))
            buf[pl.ds(vi * L, L)] = jnp.minimum(a, b)
            buf[pl.ds(wi * L, L)] = lax.rev(jnp.maximum(a, b), (0,))


def _d_pass(bufs, dv):
    sd = dv.bit_length() - 1

    @plsc.parallel_loop(0, NV // 2, unroll=4)
    def body(j):
        vi = ((j >> sd) << (sd + 1)) | (j & (dv - 1))
        for buf in bufs:
            a = buf[pl.ds(vi * L, L)]
            b = buf[pl.ds((vi + dv) * L, L)]
            buf[pl.ds(vi * L, L)] = jnp.minimum(a, b)
            buf[pl.ds((vi + dv) * L, L)] = jnp.maximum(a, b)


_SC_MESH = plsc.VectorSubcoreMesh(core_axis_name="c", subcore_axis_name="s")


@functools.partial(
    pl.kernel,
    out_type=jax.ShapeDtypeStruct((B, L), jnp.float32),
    mesh=_SC_MESH,
    compiler_params=pltpu.CompilerParams(needs_layout_passes=False),
    scratch_types=[
        pltpu.VMEM((N * D,), jnp.float32),
        pltpu.VMEM((N * D,), jnp.float32),
        pltpu.VMEM((N,), jnp.float32),
        pltpu.VMEM((N,), jnp.float32),
        pltpu.VMEM((L,), jnp.float32),
    ],
)
def _swd_sc(x_hbm, y_hbm, out_hbm, xraw, yraw, u, v, ovec):
    b = lax.axis_index("s") * 2 + lax.axis_index("c")

    pltpu.sync_copy(x_hbm.at[b], xraw)
    pltpu.sync_copy(y_hbm.at[b], yraw)

    @plsc.parallel_loop(0, NV // 2, unroll=2)
    def proj_body(g):
        g0 = g * 2
        for raw, out in ((xraw, u), (yraw, v)):
            regs = []
            for i in range(2):
                j = g0 + i
                p0 = raw[pl.ds(j * L, L)]
                p1 = raw[pl.ds(N + j * L, L)]
                p2 = raw[pl.ds(2 * N + j * L, L)]
                regs.append(_vsort((p0 + p1 + p2) * THETA))
            regs = _apply_group(regs, ["R", "s"])
            for i in range(2):
                out[pl.ds((g0 + i) * L, L)] = regs[i]

    bufs = (u, v)
    _fused_pass(bufs, 4, ["R", ("d", 1), "s"], unroll=2)
    _fused_pass(bufs, 8, ["R", ("d", 2), ("d", 1), "s"])
    tail = [("d", 4), ("d", 2), ("d", 1), "s"]
    for m in (128, 256, 512):
        mv2 = 2 * m // L
        _rev_pass(bufs, mv2)
        dv = mv2 // 4
        while dv >= 8:
            _d_pass(bufs, dv)
            dv //= 2
        _fused_pass(bufs, 8, tail)

    _rev_pass(bufs, 128)
    for dv in (32, 16, 8):
        _d_pass(bufs, dv)

    zero = jnp.zeros((L,), jnp.float32)

    @plsc.parallel_loop(0, NV // 8, carry=(zero, zero))
    def diff_acc(g, acc):
        g0 = g * 8
        ru = [u[pl.ds((g0 + i) * L, L)] for i in range(8)]
        rv = [v[pl.ds((g0 + i) * L, L)] for i in range(8)]
        ru = _apply_group(ru, tail)
        rv = _apply_group(rv, tail)
        a0, a1 = acc
        for i in range(8):
            d = ru[i] - rv[i]
            if i % 2 == 0:
                a0 = a0 + d * d
            else:
                a1 = a1 + d * d
        return (a0, a1)

    acc0, acc1 = diff_acc
    s = jnp.sum(acc0 + acc1)
    ovec[...] = jnp.broadcast_to(s, (L,))
    pltpu.sync_copy(ovec, out_hbm.at[b])


def _finish_tc(s_ref, o_ref):
    o_ref[0, 0] = jnp.sum(jnp.sqrt(s_ref[...])) * jnp.float32(1.0 / (B * L))


_finish = pl.pallas_call(
    _finish_tc,
    out_shape=jax.ShapeDtypeStruct((1, 1), jnp.float32),
    out_specs=pl.BlockSpec(memory_space=pltpu.SMEM),
)


def kernel(x, y):
    xf = x.transpose(0, 2, 1).reshape(B, D * N)
    yf = y.transpose(0, 2, 1).reshape(B, D * N)
    s = _swd_sc(xf, yf)
    return _finish(s)[0, 0]
